# baseline (speedup 1.0000x reference)
import functools

import jax
import jax.numpy as jnp
from jax import lax
from jax.experimental import pallas as pl
from jax.experimental.pallas import tpu as pltpu
from jax.experimental.pallas import tpu_sc as plsc

E = 16
D_MODEL = 1024
D_FF = 2048
N_TOK = 2048
BP = 256
CH = 128
G = N_TOK // BP + E
P = G * BP


def _router_body(x_ref, wg_ref, pos_ref, te_ref, row_ref, valid_ref):
    x = x_ref[...]
    wg = wg_ref[...]
    logits = lax.dot_general(x, wg, (((1,), (1,)), ((), ())),
                             preferred_element_type=jnp.float32)
    maxv = jnp.max(logits, axis=1, keepdims=True)
    col = lax.broadcasted_iota(jnp.int32, logits.shape, 1)
    cand = jnp.where(logits == maxv, col, jnp.int32(E))
    eid = jnp.min(cand, axis=1, keepdims=True)
    mask = (col == eid).astype(jnp.float32)

    counts = jnp.sum(mask, axis=0, keepdims=True)

    nblk = jnp.ceil(counts * (1.0 / BP))
    ei = lax.broadcasted_iota(jnp.int32, (E, E), 0)
    ej = lax.broadcasted_iota(jnp.int32, (E, E), 1)
    lt_e = (ei < ej).astype(jnp.float32)
    tstart = lax.dot_general(nblk, lt_e, (((1,), (0,)), ((), ())),
                             preferred_element_type=jnp.float32)
    run = tstart * BP

    tend = tstart + nblk
    total = jnp.sum(nblk, keepdims=True)
    gi = lax.broadcasted_iota(jnp.int32, (G, 1), 0).astype(jnp.float32)
    gclamp = jnp.minimum(gi, total - 1.0)
    te = jnp.sum((gclamp >= tend).astype(jnp.float32), axis=1, keepdims=True)
    te_ref[...] = jnp.minimum(te, float(E - 1)).astype(jnp.int32)
    row_ref[...] = gclamp.astype(jnp.int32)
    valid_ref[...] = (gi < total).astype(jnp.int32)

    ri = lax.broadcasted_iota(jnp.int32, (CH, CH), 0)
    rj = lax.broadcasted_iota(jnp.int32, (CH, CH), 1)
    lt_c = (rj < ri).astype(jnp.float32)
    for i in range(N_TOK // CH):
        m = mask[i * CH:(i + 1) * CH, :]
        rm = lax.dot_general(lt_c, m, (((1,), (0,)), ((), ())),
                             preferred_element_type=jnp.float32) + run
        p = jnp.sum(rm * m, axis=1, keepdims=True)
        pos_ref[pl.ds(i * CH, CH), :] = p.astype(jnp.int32)
        run = run + jnp.sum(m, axis=0, keepdims=True)


def _route(xs, Wg):
    return pl.pallas_call(
        _router_body,
        out_shape=(
            jax.ShapeDtypeStruct((N_TOK, 1), jnp.int32),
            jax.ShapeDtypeStruct((G, 1), jnp.int32),
            jax.ShapeDtypeStruct((G, 1), jnp.int32),
            jax.ShapeDtypeStruct((G, 1), jnp.int32),
        ),
    )(xs, Wg)


DFH = D_FF // 2


def _ffn_body(te_ref, row_ref, valid_ref, x_ref,
              w1a_ref, w1b_ref, w3a_ref, w3b_ref, w2a_ref, w2b_ref, o_ref):
    g = pl.program_id(0)

    @pl.when(valid_ref[g] == 1)
    def _():
        xb = x_ref[...]

        def half(w1_ref, w3_ref, w2_ref):
            a = lax.dot_general(xb, w1_ref[0], (((1,), (1,)), ((), ())),
                                preferred_element_type=jnp.float32)
            b = lax.dot_general(xb, w3_ref[0], (((1,), (1,)), ((), ())),
                                preferred_element_type=jnp.float32)
            h = a / (1.0 + jnp.exp(-a)) * b
            return lax.dot_general(h, w2_ref[0], (((1,), (1,)), ((), ())),
                                   preferred_element_type=jnp.float32)

        o_ref[...] = (half(w1a_ref, w3a_ref, w2a_ref)
                      + half(w1b_ref, w3b_ref, w2b_ref))


def _ffn(te, row, valid, x_sorted, W1, W3, W2):
    def _wa(g, te, row, valid):
        return (te[g], 0, 0)

    def _wb(g, te, row, valid):
        return (te[g], 1, 0)

    def _w2b(g, te, row, valid):
        return (te[g], 0, 1)

    grid_spec = pltpu.PrefetchScalarGridSpec(
        num_scalar_prefetch=3,
        grid=(G,),
        in_specs=[
            pl.BlockSpec((BP, D_MODEL), lambda g, te, row, valid: (row[g], 0)),
            pl.BlockSpec((1, DFH, D_MODEL), _wa),
            pl.BlockSpec((1, DFH, D_MODEL), _wb),
            pl.BlockSpec((1, DFH, D_MODEL), _wa),
            pl.BlockSpec((1, DFH, D_MODEL), _wb),
            pl.BlockSpec((1, D_MODEL, DFH), _wa),
            pl.BlockSpec((1, D_MODEL, DFH), _w2b),
        ],
        out_specs=pl.BlockSpec((BP, D_MODEL),
                               lambda g, te, row, valid: (row[g], 0)),
    )
    return pl.pallas_call(
        _ffn_body,
        grid_spec=grid_spec,
        out_shape=jax.ShapeDtypeStruct((P, D_MODEL), jnp.float32),
    )(te, row, valid, x_sorted, W1, W1, W3, W3, W2, W2)


@functools.lru_cache(maxsize=None)
def _sc_kernels():
    info = plsc.get_sparse_core_info()
    nc, ns = info.num_cores, info.num_subcores
    nw = nc * ns
    rpw = N_TOK // nw
    mesh = plsc.VectorSubcoreMesh(core_axis_name="c", subcore_axis_name="s")
    scratch = [
        pltpu.VMEM((rpw,), jnp.int32),
        pltpu.VMEM((rpw, D_MODEL), jnp.float32),
        pltpu.SemaphoreType.DMA,
    ]

    @functools.partial(
        pl.kernel, mesh=mesh,
        out_type=jax.ShapeDtypeStruct((P, D_MODEL), jnp.float32),
        scratch_types=scratch,
    )
    def dispatch(x_hbm, pos_hbm, out_hbm, idx_v, rows_v, sem):
        wid = lax.axis_index("s") * nc + lax.axis_index("c")
        base = wid * rpw
        pltpu.sync_copy(pos_hbm.at[pl.ds(base, rpw)], idx_v)
        pltpu.sync_copy(x_hbm.at[pl.ds(base, rpw)], rows_v)
        pltpu.async_copy(rows_v, out_hbm.at[idx_v], sem).wait()

    @functools.partial(
        pl.kernel, mesh=mesh,
        out_type=jax.ShapeDtypeStruct((N_TOK, D_MODEL), jnp.float32),
        scratch_types=scratch,
    )
    def undispatch(tab_hbm, pos_hbm, out_hbm, idx_v, rows_v, sem):
        wid = lax.axis_index("s") * nc + lax.axis_index("c")
        base = wid * rpw
        pltpu.sync_copy(pos_hbm.at[pl.ds(base, rpw)], idx_v)
        pltpu.async_copy(tab_hbm.at[idx_v], rows_v, sem).wait()
        pltpu.sync_copy(rows_v, out_hbm.at[pl.ds(base, rpw)])

    return dispatch, undispatch


def kernel(x, Wg, W1, W2, W3):
    xs = x.reshape(-1, x.shape[-1])
    pos2d, te2d, row2d, valid2d = _route(xs, Wg)
    pos = pos2d.reshape(-1)
    te = te2d.reshape(-1)
    row = row2d.reshape(-1)
    valid = valid2d.reshape(-1)

    dispatch, undispatch = _sc_kernels()
    x_sorted = dispatch(xs, pos)
    out_sorted = _ffn(te, row, valid, x_sorted, W1, W3, W2)
    out = undispatch(out_sorted, pos)
    return out.reshape(x.shape)


# speedup vs baseline: 1.0021x; 1.0021x over previous
import functools

import jax
import jax.numpy as jnp
from jax import lax
from jax.experimental import pallas as pl
from jax.experimental.pallas import tpu as pltpu
from jax.experimental.pallas import tpu_sc as plsc

E = 16
D_MODEL = 1024
D_FF = 2048
N_TOK = 2048
BP = 256
CH = 128
G = N_TOK // BP + E
P = G * BP


def _router_body(x_ref, wg_ref, pos_ref, te_ref, row_ref, valid_ref):
    x = x_ref[...]
    wg = wg_ref[...]
    logits = lax.dot_general(x, wg, (((1,), (1,)), ((), ())),
                             preferred_element_type=jnp.float32)
    maxv = jnp.max(logits, axis=1, keepdims=True)
    col = lax.broadcasted_iota(jnp.int32, logits.shape, 1)
    cand = jnp.where(logits == maxv, col, jnp.int32(E))
    eid = jnp.min(cand, axis=1, keepdims=True)
    mask = (col == eid).astype(jnp.float32)

    counts = jnp.sum(mask, axis=0, keepdims=True)

    nblk = jnp.ceil(counts * (1.0 / BP))
    ei = lax.broadcasted_iota(jnp.int32, (E, E), 0)
    ej = lax.broadcasted_iota(jnp.int32, (E, E), 1)
    lt_e = (ei < ej).astype(jnp.float32)
    tstart = lax.dot_general(nblk, lt_e, (((1,), (0,)), ((), ())),
                             preferred_element_type=jnp.float32)
    run = tstart * BP

    tend = tstart + nblk
    total = jnp.sum(nblk, keepdims=True)
    gi = lax.broadcasted_iota(jnp.int32, (G, 1), 0).astype(jnp.float32)
    gclamp = jnp.minimum(gi, total - 1.0)
    te = jnp.sum((gclamp >= tend).astype(jnp.float32), axis=1, keepdims=True)
    te_ref[...] = jnp.minimum(te, float(E - 1)).astype(jnp.int32)
    row_ref[...] = gclamp.astype(jnp.int32)
    valid_ref[...] = (gi < total).astype(jnp.int32)

    ri = lax.broadcasted_iota(jnp.int32, (CH, CH), 0)
    rj = lax.broadcasted_iota(jnp.int32, (CH, CH), 1)
    lt_c = (rj < ri).astype(jnp.float32)
    for i in range(N_TOK // CH):
        m = mask[i * CH:(i + 1) * CH, :]
        rm = lax.dot_general(lt_c, m, (((1,), (0,)), ((), ())),
                             preferred_element_type=jnp.float32) + run
        p = jnp.sum(rm * m, axis=1, keepdims=True)
        pos_ref[pl.ds(i * CH, CH), :] = p.astype(jnp.int32)
        run = run + jnp.sum(m, axis=0, keepdims=True)


def _route(xs, Wg):
    return pl.pallas_call(
        _router_body,
        out_shape=(
            jax.ShapeDtypeStruct((N_TOK, 1), jnp.int32),
            jax.ShapeDtypeStruct((G, 1), jnp.int32),
            jax.ShapeDtypeStruct((G, 1), jnp.int32),
            jax.ShapeDtypeStruct((G, 1), jnp.int32),
        ),
    )(xs, Wg)


def _ffn_body(te_ref, row_ref, valid_ref, x_ref, w1_ref, w3_ref, w2_ref, o_ref):
    g = pl.program_id(0)

    @pl.when(valid_ref[g] == 1)
    def _():
        xb = x_ref[...]
        w1 = w1_ref[0]
        w3 = w3_ref[0]
        w2 = w2_ref[0]
        a = lax.dot_general(xb, w1, (((1,), (1,)), ((), ())),
                            preferred_element_type=jnp.float32)
        b = lax.dot_general(xb, w3, (((1,), (1,)), ((), ())),
                            preferred_element_type=jnp.float32)
        h = a / (1.0 + jnp.exp(-a)) * b
        o_ref[...] = lax.dot_general(h, w2, (((1,), (1,)), ((), ())),
                                     preferred_element_type=jnp.float32)


def _ffn(te, row, valid, x_sorted, W1, W3, W2):
    grid_spec = pltpu.PrefetchScalarGridSpec(
        num_scalar_prefetch=3,
        grid=(G,),
        in_specs=[
            pl.BlockSpec((BP, D_MODEL), lambda g, te, row, valid: (row[g], 0)),
            pl.BlockSpec((1, D_FF, D_MODEL),
                         lambda g, te, row, valid: (te[g], 0, 0)),
            pl.BlockSpec((1, D_FF, D_MODEL),
                         lambda g, te, row, valid: (te[g], 0, 0)),
            pl.BlockSpec((1, D_MODEL, D_FF),
                         lambda g, te, row, valid: (te[g], 0, 0)),
        ],
        out_specs=pl.BlockSpec((BP, D_MODEL),
                               lambda g, te, row, valid: (row[g], 0)),
    )
    return pl.pallas_call(
        _ffn_body,
        grid_spec=grid_spec,
        out_shape=jax.ShapeDtypeStruct((P, D_MODEL), jnp.float32),
    )(te, row, valid, x_sorted, W1, W3, W2)


@functools.lru_cache(maxsize=None)
def _sc_kernels():
    info = plsc.get_sparse_core_info()
    nc, ns = info.num_cores, info.num_subcores
    nw = nc * ns
    rpw = N_TOK // nw
    mesh = plsc.VectorSubcoreMesh(core_axis_name="c", subcore_axis_name="s")
    scratch = [
        pltpu.VMEM((rpw,), jnp.int32),
        pltpu.VMEM((rpw, D_MODEL), jnp.float32),
        pltpu.SemaphoreType.DMA,
    ]

    @functools.partial(
        pl.kernel, mesh=mesh,
        out_type=jax.ShapeDtypeStruct((P, D_MODEL), jnp.float32),
        scratch_types=scratch,
    )
    def dispatch(x_hbm, pos_hbm, out_hbm, idx_v, rows_v, sem):
        wid = lax.axis_index("s") * nc + lax.axis_index("c")
        base = wid * rpw
        pltpu.sync_copy(pos_hbm.at[pl.ds(base, rpw)], idx_v)
        pltpu.sync_copy(x_hbm.at[pl.ds(base, rpw)], rows_v)
        pltpu.async_copy(rows_v, out_hbm.at[idx_v], sem).wait()

    @functools.partial(
        pl.kernel, mesh=mesh,
        out_type=jax.ShapeDtypeStruct((N_TOK, D_MODEL), jnp.float32),
        scratch_types=scratch,
    )
    def undispatch(tab_hbm, pos_hbm, out_hbm, idx_v, rows_v, sem):
        wid = lax.axis_index("s") * nc + lax.axis_index("c")
        base = wid * rpw
        pltpu.sync_copy(pos_hbm.at[pl.ds(base, rpw)], idx_v)
        pltpu.async_copy(tab_hbm.at[idx_v], rows_v, sem).wait()
        pltpu.sync_copy(rows_v, out_hbm.at[pl.ds(base, rpw)])

    return dispatch, undispatch


def kernel(x, Wg, W1, W2, W3):
    xs = x.reshape(-1, x.shape[-1])
    pos2d, te2d, row2d, valid2d = _route(xs, Wg)
    pos = pos2d.reshape(-1)
    te = te2d.reshape(-1)
    row = row2d.reshape(-1)
    valid = valid2d.reshape(-1)

    dispatch, undispatch = _sc_kernels()
    x_sorted = dispatch(xs, pos)
    out_sorted = _ffn(te, row, valid, x_sorted, W1, W3, W2)
    out = undispatch(out_sorted, pos)
    return out.reshape(x.shape)


# speedup vs baseline: 1.0397x; 1.0376x over previous
import functools

import jax
import jax.numpy as jnp
from jax import lax
from jax.experimental import pallas as pl
from jax.experimental.pallas import tpu as pltpu
from jax.experimental.pallas import tpu_sc as plsc

E = 16
D_MODEL = 1024
D_FF = 2048
N_TOK = 2048
BP = 256
CH = 128
G = N_TOK // BP + E
P = G * BP


def _router_body(x_ref, wg_ref, pos_ref, te_ref, row_ref, valid_ref):
    x = x_ref[...]
    wg = wg_ref[...]
    logits = lax.dot_general(x, wg, (((1,), (1,)), ((), ())),
                             preferred_element_type=jnp.float32)
    maxv = jnp.max(logits, axis=1, keepdims=True)
    col = lax.broadcasted_iota(jnp.int32, logits.shape, 1)
    cand = jnp.where(logits == maxv, col, jnp.int32(E))
    eid = jnp.min(cand, axis=1, keepdims=True)
    mask = (col == eid).astype(jnp.float32)

    counts = jnp.sum(mask, axis=0, keepdims=True)

    nblk = jnp.ceil(counts * (1.0 / BP))
    ei = lax.broadcasted_iota(jnp.int32, (E, E), 0)
    ej = lax.broadcasted_iota(jnp.int32, (E, E), 1)
    lt_e = (ei < ej).astype(jnp.float32)
    tstart = lax.dot_general(nblk, lt_e, (((1,), (0,)), ((), ())),
                             preferred_element_type=jnp.float32)
    run = tstart * BP

    tend = tstart + nblk
    total = jnp.sum(nblk, keepdims=True)
    gi = lax.broadcasted_iota(jnp.int32, (G, 1), 0).astype(jnp.float32)
    gclamp = jnp.minimum(gi, total - 1.0)
    te = jnp.sum((gclamp >= tend).astype(jnp.float32), axis=1, keepdims=True)
    te_ref[...] = jnp.minimum(te, float(E - 1)).astype(jnp.int32)
    row_ref[...] = gclamp.astype(jnp.int32)
    valid_ref[...] = (gi < total).astype(jnp.int32)

    ri = lax.broadcasted_iota(jnp.int32, (CH, CH), 0)
    rj = lax.broadcasted_iota(jnp.int32, (CH, CH), 1)
    lt_c = (rj < ri).astype(jnp.float32)
    for i in range(N_TOK // CH):
        m = mask[i * CH:(i + 1) * CH, :]
        rm = lax.dot_general(lt_c, m, (((1,), (0,)), ((), ())),
                             preferred_element_type=jnp.float32) + run
        p = jnp.sum(rm * m, axis=1, keepdims=True)
        pos_ref[pl.ds(i * CH, CH), :] = p.astype(jnp.int32)
        run = run + jnp.sum(m, axis=0, keepdims=True)


def _route(xs, Wg):
    return pl.pallas_call(
        _router_body,
        out_shape=(
            jax.ShapeDtypeStruct((N_TOK, 1), jnp.int32),
            jax.ShapeDtypeStruct((G, 1), jnp.int32),
            jax.ShapeDtypeStruct((G, 1), jnp.int32),
            jax.ShapeDtypeStruct((G, 1), jnp.int32),
        ),
    )(xs, Wg)


def _ffn_body(te_ref, row_ref, valid_ref, pos_ref, x_ref,
              w1_ref, w3_ref, w2_ref, o_ref):
    g = pl.program_id(0)

    @pl.when(valid_ref[g] == 1)
    def _():
        pos = pos_ref[...]
        target = (row_ref[g] * BP
                  + lax.broadcasted_iota(jnp.int32, (BP, N_TOK), 0))
        sel = (pos == target).astype(jnp.float32)
        xb = lax.dot_general(sel, x_ref[...], (((1,), (0,)), ((), ())),
                             preferred_element_type=jnp.float32)
        w1 = w1_ref[0]
        w3 = w3_ref[0]
        w2 = w2_ref[0]
        a = lax.dot_general(xb, w1, (((1,), (1,)), ((), ())),
                            preferred_element_type=jnp.float32)
        b = lax.dot_general(xb, w3, (((1,), (1,)), ((), ())),
                            preferred_element_type=jnp.float32)
        h = a / (1.0 + jnp.exp(-a)) * b
        o_ref[...] = lax.dot_general(h, w2, (((1,), (1,)), ((), ())),
                                     preferred_element_type=jnp.float32)


def _ffn(te, row, valid, pos_row, xs, W1, W3, W2):
    grid_spec = pltpu.PrefetchScalarGridSpec(
        num_scalar_prefetch=3,
        grid=(G,),
        in_specs=[
            pl.BlockSpec((1, N_TOK), lambda g, te, row, valid: (0, 0)),
            pl.BlockSpec((N_TOK, D_MODEL), lambda g, te, row, valid: (0, 0)),
            pl.BlockSpec((1, D_FF, D_MODEL),
                         lambda g, te, row, valid: (te[g], 0, 0)),
            pl.BlockSpec((1, D_FF, D_MODEL),
                         lambda g, te, row, valid: (te[g], 0, 0)),
            pl.BlockSpec((1, D_MODEL, D_FF),
                         lambda g, te, row, valid: (te[g], 0, 0)),
        ],
        out_specs=pl.BlockSpec((BP, D_MODEL),
                               lambda g, te, row, valid: (row[g], 0)),
    )
    return pl.pallas_call(
        _ffn_body,
        grid_spec=grid_spec,
        out_shape=jax.ShapeDtypeStruct((P, D_MODEL), jnp.float32),
        compiler_params=pltpu.CompilerParams(
            vmem_limit_bytes=128 * 1024 * 1024),
    )(te, row, valid, pos_row, xs, W1, W3, W2)


@functools.lru_cache(maxsize=None)
def _sc_kernels():
    info = plsc.get_sparse_core_info()
    nc, ns = info.num_cores, info.num_subcores
    nw = nc * ns
    rpw = N_TOK // nw
    mesh = plsc.VectorSubcoreMesh(core_axis_name="c", subcore_axis_name="s")
    scratch = [
        pltpu.VMEM((rpw,), jnp.int32),
        pltpu.VMEM((rpw, D_MODEL), jnp.float32),
        pltpu.SemaphoreType.DMA,
    ]

    @functools.partial(
        pl.kernel, mesh=mesh,
        out_type=jax.ShapeDtypeStruct((P, D_MODEL), jnp.float32),
        scratch_types=scratch,
    )
    def dispatch(x_hbm, pos_hbm, out_hbm, idx_v, rows_v, sem):
        wid = lax.axis_index("s") * nc + lax.axis_index("c")
        base = wid * rpw
        pltpu.sync_copy(pos_hbm.at[pl.ds(base, rpw)], idx_v)
        pltpu.sync_copy(x_hbm.at[pl.ds(base, rpw)], rows_v)
        pltpu.async_copy(rows_v, out_hbm.at[idx_v], sem).wait()

    @functools.partial(
        pl.kernel, mesh=mesh,
        out_type=jax.ShapeDtypeStruct((N_TOK, D_MODEL), jnp.float32),
        scratch_types=scratch,
    )
    def undispatch(tab_hbm, pos_hbm, out_hbm, idx_v, rows_v, sem):
        wid = lax.axis_index("s") * nc + lax.axis_index("c")
        base = wid * rpw
        pltpu.sync_copy(pos_hbm.at[pl.ds(base, rpw)], idx_v)
        pltpu.async_copy(tab_hbm.at[idx_v], rows_v, sem).wait()
        pltpu.sync_copy(rows_v, out_hbm.at[pl.ds(base, rpw)])

    return dispatch, undispatch


def kernel(x, Wg, W1, W2, W3):
    xs = x.reshape(-1, x.shape[-1])
    pos2d, te2d, row2d, valid2d = _route(xs, Wg)
    pos = pos2d.reshape(-1)
    te = te2d.reshape(-1)
    row = row2d.reshape(-1)
    valid = valid2d.reshape(-1)

    undispatch = _sc_kernels()[1]
    pos_row = pos2d.reshape(1, N_TOK)
    out_sorted = _ffn(te, row, valid, pos_row, xs, W1, W3, W2)
    out = undispatch(out_sorted, pos)
    return out.reshape(x.shape)
